# baseline (speedup 1.0000x reference)
import functools

import jax
import jax.numpy as jnp
from jax import lax
from jax.experimental import pallas as pl
from jax.experimental.pallas import tpu as pltpu
from jax.experimental.pallas import tpu_sc as plsc

N_NODES = 10000
N_EDGES = 320000
D = 128
R = 4

NC = 2
NS = 16
NW = NC * NS
L = 16
EPT = N_EDGES // NW
CH = 80
NCH = EPT // CH
G = CH // L
KV = D // L
WB_STRIDE = 624
WB_ROWS = 640
NSUP = 5
SUBCH = NCH // NSUP

_GDN = lax.GatherDimensionNumbers(
    offset_dims=(), collapsed_slice_dims=(0,), start_index_map=(0,))


def _lane_bcast(v, t):
    idx = jnp.full((L, 1), t, jnp.int32)
    return lax.gather(v, idx, dimension_numbers=_GDN, slice_sizes=(1,),
                      mode=lax.GatherScatterMode.PROMISE_IN_BOUNDS)


def _sc_mesh():
    return plsc.VectorSubcoreMesh(
        core_axis_name="c", subcore_axis_name="s",
        num_cores=NC, num_subcores=NS)


def _mm_body(x_ref, w_ref, o_ref):
    o_ref[0] = jnp.dot(x_ref[...], w_ref[0], preferred_element_type=jnp.float32)


def _compute_h(x, W):
    BM = 1000
    return pl.pallas_call(
        _mm_body,
        grid=(R, N_NODES // BM),
        in_specs=[
            pl.BlockSpec((BM, D), lambda r, i: (i, 0)),
            pl.BlockSpec((1, D, D), lambda r, i: (r, 0, 0)),
        ],
        out_specs=pl.BlockSpec((1, BM, D), lambda r, i: (r, i, 0)),
        out_shape=jax.ShapeDtypeStruct((R, N_NODES, D), jnp.float32),
    )(x, W)


def _pass1(h_flat, isrc3, idst3):
    @functools.partial(
        pl.kernel,
        out_type=jax.ShapeDtypeStruct((NW, NCH, CH), jnp.float32),
        mesh=_sc_mesh(),
        scratch_types=[
            pltpu.VMEM((NCH, CH), jnp.int32),
            pltpu.VMEM((NCH, CH), jnp.int32),
            pltpu.VMEM((3, CH, D), jnp.float32),
            pltpu.VMEM((3, CH, D), jnp.float32),
            pltpu.VMEM((NCH, CH), jnp.float32),
            pltpu.VMEM((L * 17,), jnp.float32),
            pltpu.SemaphoreType.DMA,
            pltpu.SemaphoreType.DMA,
            pltpu.SemaphoreType.DMA,
            pltpu.SemaphoreType.DMA,
            pltpu.SemaphoreType.DMA,
            pltpu.SemaphoreType.DMA,
        ],
        compiler_params=pltpu.CompilerParams(needs_layout_passes=False),
    )
    def k(h_hbm, isrc_hbm, idst_hbm, out_hbm,
          isrc_v, idst_v, rs_v, rd_v, lg_v, tr_v,
          s0, d0, s1, d1, s2, d2):
        wid = lax.axis_index("c") * NS + lax.axis_index("s")
        pltpu.sync_copy(isrc_hbm.at[wid], isrc_v)
        pltpu.sync_copy(idst_hbm.at[wid], idst_v)
        lanes = lax.iota(jnp.int32, L)
        sems = ((s0, d0), (s1, d1), (s2, d2))
        NB = 3

        def start(j, b):
            ss, sd = sems[b]
            pltpu.async_copy(h_hbm.at[isrc_v.at[j]], rs_v.at[b], ss)
            pltpu.async_copy(h_hbm.at[idst_v.at[j]], rd_v.at[b], sd)

        def wait(b):
            ss, sd = sems[b]
            pltpu.make_async_copy(
                h_hbm.at[pl.ds(0, CH)], rs_v.at[b], ss).wait()
            pltpu.make_async_copy(
                h_hbm.at[pl.ds(0, CH)], rd_v.at[b], sd).wait()

        def compute(j, b):
            def group(g, carry):
                for t in range(L):
                    e = g * L + t
                    acc = rs_v[b, e, pl.ds(0, L)] * rd_v[b, e, pl.ds(0, L)]
                    for kk in range(1, KV):
                        acc = acc + (rs_v[b, e, pl.ds(kk * L, L)]
                                     * rd_v[b, e, pl.ds(kk * L, L)])
                    plsc.store_scatter(tr_v, [lanes * 17 + t], acc)
                tot = plsc.load_gather(tr_v, [lanes])
                for l in range(1, L):
                    tot = tot + plsc.load_gather(tr_v, [lanes + l * 17])
                tot = jnp.where(tot > 0, tot, 0.2 * tot)
                lg_v[j, pl.ds(g * L, L)] = tot
                return carry

            lax.fori_loop(0, G, group, 0)

        for b in range(NB):
            start(b, b)

        def body(jo, carry):
            j0 = jo * NB
            for b in range(NB):
                j = j0 + b
                wait(b)
                compute(j, b)

                @pl.when(j + NB < NCH)
                def _():
                    start(j + NB, b)

            return carry

        lax.fori_loop(0, NCH // NB, body, 0)
        for b in range(NCH % NB):
            wait(b)
            compute(NCH - (NCH % NB) + b, b)
        pltpu.sync_copy(lg_v, out_hbm.at[wid])

    return k(h_flat, isrc3, idst3)


def _softmax_body(lg_ref, et_ref, o_ref):
    lg = lg_ref[...]
    et = et_ref[...]
    acc = jnp.zeros_like(lg)
    for r in range(R):
        m = et == r
        mr = jnp.max(jnp.where(m, lg, -jnp.inf))
        e = jnp.where(m, jnp.exp(lg - mr), 0.0)
        sr = jnp.sum(e)
        acc = acc + jnp.where(m, e / sr, 0.0)
    o_ref[...] = acc


def _softmax(logits2, et2):
    return pl.pallas_call(
        _softmax_body,
        out_shape=jax.ShapeDtypeStruct(logits2.shape, jnp.float32),
    )(logits2, et2)


def _pass2(h_flat, isrc3, dst3, attn3):
    @functools.partial(
        pl.kernel,
        out_type=jax.ShapeDtypeStruct((NC, N_NODES, D), jnp.float32),
        mesh=_sc_mesh(),
        scratch_types=[
            pltpu.VMEM((SUBCH, CH), jnp.int32),
            pltpu.VMEM((SUBCH, CH), jnp.int32),
            pltpu.VMEM((SUBCH, CH), jnp.float32),
            pltpu.VMEM((3, CH, D), jnp.float32),
            pltpu.VMEM_SHARED((N_NODES, D), jnp.float32),
            pltpu.SemaphoreType.DMA,
            pltpu.SemaphoreType.DMA,
            pltpu.SemaphoreType.DMA,
        ],
        compiler_params=pltpu.CompilerParams(needs_layout_passes=False),
    )
    def k(h_hbm, isrc_hbm, dst_hbm, attn_hbm, out_hbm,
          isrc_v, dst_v, attn_v, rows_v, acc_s, g0, g1, g2):
        cid = lax.axis_index("c")
        sid = lax.axis_index("s")
        wid = cid * NS + sid
        gsems = (g0, g1, g2)
        NB = 3

        zv = jnp.zeros((L,), jnp.float32)

        def zrow(i, carry):
            for kk in range(KV):
                rows_v[0, i, pl.ds(kk * L, L)] = zv
            return carry

        lax.fori_loop(0, CH, zrow, 0)
        for b in range(WB_ROWS // CH):
            pltpu.sync_copy(
                rows_v.at[0], acc_s.at[pl.ds(sid * WB_STRIDE + b * CH, CH)])
        plsc.subcore_barrier()

        def g_start(j, b):
            pltpu.async_copy(h_hbm.at[isrc_v.at[j]], rows_v.at[b], gsems[b])

        def g_wait(b):
            pltpu.make_async_copy(
                h_hbm.at[pl.ds(0, CH)], rows_v.at[b], gsems[b]).wait()

        def scale(j, b):
            jv = jnp.full((L,), j, jnp.int32)

            def group(g, carry):
                for t in range(L):
                    e = g * L + t
                    ae = plsc.load_gather(
                        attn_v, [jv, jnp.full((L,), e, jnp.int32)])
                    for kk in range(KV):
                        rows_v[b, e, pl.ds(kk * L, L)] = (
                            rows_v[b, e, pl.ds(kk * L, L)] * ae)
                return carry

            lax.fori_loop(0, G, group, 0)

        def supchunk(s, carry):
            pltpu.sync_copy(isrc_hbm.at[wid, s], isrc_v)
            pltpu.sync_copy(dst_hbm.at[wid, s], dst_v)
            pltpu.sync_copy(attn_hbm.at[wid, s], attn_v)
            for b in range(NB):
                g_start(b, b)

            def body(jo, carry2):
                j0 = jo * NB
                for b in range(NB):
                    j = j0 + b
                    g_wait(b)
                    scale(j, b)
                    pltpu.sync_copy(
                        rows_v.at[b], acc_s.at[dst_v.at[j]], add=True)

                    @pl.when(j + NB < SUBCH)
                    def _():
                        g_start(j + NB, b)

                return carry2

            lax.fori_loop(0, SUBCH // NB, body, 0)
            g_wait(0)
            scale(SUBCH - 1, 0)
            pltpu.sync_copy(
                rows_v.at[0], acc_s.at[dst_v.at[SUBCH - 1]], add=True)
            return carry

        lax.fori_loop(0, NSUP, supchunk, 0)
        plsc.subcore_barrier()
        pltpu.sync_copy(
            acc_s.at[pl.ds(sid * WB_STRIDE, WB_ROWS)],
            out_hbm.at[cid, pl.ds(sid * WB_STRIDE, WB_ROWS)])

    return k(h_flat, isrc3, dst3, attn3)


def _add_body(a_ref, b_ref, o_ref):
    o_ref[...] = a_ref[...] + b_ref[...]


def _final_add(a, b):
    BM = 1000
    return pl.pallas_call(
        _add_body,
        grid=(N_NODES // BM,),
        in_specs=[
            pl.BlockSpec((BM, D), lambda i: (i, 0)),
            pl.BlockSpec((BM, D), lambda i: (i, 0)),
        ],
        out_specs=pl.BlockSpec((BM, D), lambda i: (i, 0)),
        out_shape=jax.ShapeDtypeStruct((N_NODES, D), jnp.float32),
    )(a, b)


def kernel(x, edge_index, edge_type, W):
    src = edge_index[0]
    dst = edge_index[1]
    flat_src = (edge_type * N_NODES + src).reshape(NW, NCH, CH)
    flat_dst = (edge_type * N_NODES + dst).reshape(NW, NCH, CH)
    dst3 = dst.reshape(NW, NCH, CH)

    h = _compute_h(x, W).reshape(R * N_NODES, D)
    logits = _pass1(h, flat_src, flat_dst).reshape(N_EDGES)
    attn = _softmax(
        logits.reshape(N_EDGES // D, D),
        edge_type.reshape(N_EDGES // D, D),
    ).reshape(NW, NSUP, SUBCH, CH)
    parts = _pass2(h, flat_src.reshape(NW, NSUP, SUBCH, CH),
                   dst3.reshape(NW, NSUP, SUBCH, CH), attn)
    return _final_add(parts[0], parts[1])


# speedup vs baseline: 1.1108x; 1.1108x over previous
import functools

import jax
import jax.numpy as jnp
from jax import lax
from jax.experimental import pallas as pl
from jax.experimental.pallas import tpu as pltpu
from jax.experimental.pallas import tpu_sc as plsc

N_NODES = 10000
N_EDGES = 320000
D = 128
R = 4

NC = 2
NS = 16
NW = NC * NS
L = 16
EPT = N_EDGES // NW
CH = 80
NCH = EPT // CH
G = CH // L
KV = D // L
WB_STRIDE = 624
WB_ROWS = 640
NSUP = 5
SUBCH = NCH // NSUP

_GDN = lax.GatherDimensionNumbers(
    offset_dims=(), collapsed_slice_dims=(0,), start_index_map=(0,))


def _lane_bcast(v, t):
    idx = jnp.full((L, 1), t, jnp.int32)
    return lax.gather(v, idx, dimension_numbers=_GDN, slice_sizes=(1,),
                      mode=lax.GatherScatterMode.PROMISE_IN_BOUNDS)


def _sc_mesh():
    return plsc.VectorSubcoreMesh(
        core_axis_name="c", subcore_axis_name="s",
        num_cores=NC, num_subcores=NS)


def _mm_body(x_ref, w_ref, o_ref):
    o_ref[0] = jnp.dot(x_ref[...], w_ref[0], preferred_element_type=jnp.float32)


def _compute_h(x, W):
    BM = 1000
    return pl.pallas_call(
        _mm_body,
        grid=(R, N_NODES // BM),
        in_specs=[
            pl.BlockSpec((BM, D), lambda r, i: (i, 0)),
            pl.BlockSpec((1, D, D), lambda r, i: (r, 0, 0)),
        ],
        out_specs=pl.BlockSpec((1, BM, D), lambda r, i: (r, i, 0)),
        out_shape=jax.ShapeDtypeStruct((R, N_NODES, D), jnp.float32),
    )(x, W)


def _pass1(h_flat, isrc3, idst3):
    @functools.partial(
        pl.kernel,
        out_type=jax.ShapeDtypeStruct((NW, NCH, CH), jnp.float32),
        mesh=_sc_mesh(),
        scratch_types=[
            pltpu.VMEM((NCH, CH), jnp.int32),
            pltpu.VMEM((NCH, CH), jnp.int32),
            pltpu.VMEM((3, CH, D), jnp.float32),
            pltpu.VMEM((3, CH, D), jnp.float32),
            pltpu.VMEM((NCH, CH), jnp.float32),
            pltpu.VMEM((L * 17,), jnp.float32),
            pltpu.SemaphoreType.DMA,
            pltpu.SemaphoreType.DMA,
            pltpu.SemaphoreType.DMA,
            pltpu.SemaphoreType.DMA,
            pltpu.SemaphoreType.DMA,
            pltpu.SemaphoreType.DMA,
        ],
        compiler_params=pltpu.CompilerParams(needs_layout_passes=False),
    )
    def k(h_hbm, isrc_hbm, idst_hbm, out_hbm,
          isrc_v, idst_v, rs_v, rd_v, lg_v, tr_v,
          s0, d0, s1, d1, s2, d2):
        wid = lax.axis_index("c") * NS + lax.axis_index("s")
        pltpu.sync_copy(isrc_hbm.at[wid], isrc_v)
        pltpu.sync_copy(idst_hbm.at[wid], idst_v)
        lanes = lax.iota(jnp.int32, L)
        sems = ((s0, d0), (s1, d1), (s2, d2))
        NB = 3

        def start(j, b):
            ss, sd = sems[b]
            pltpu.async_copy(h_hbm.at[isrc_v.at[j]], rs_v.at[b], ss)
            pltpu.async_copy(h_hbm.at[idst_v.at[j]], rd_v.at[b], sd)

        def wait(b):
            ss, sd = sems[b]
            pltpu.make_async_copy(
                h_hbm.at[pl.ds(0, CH)], rs_v.at[b], ss).wait()
            pltpu.make_async_copy(
                h_hbm.at[pl.ds(0, CH)], rd_v.at[b], sd).wait()

        def compute(j, b):
            def group(g, carry):
                for t in range(L):
                    e = g * L + t
                    acc = rs_v[b, e, pl.ds(0, L)] * rd_v[b, e, pl.ds(0, L)]
                    for kk in range(1, KV):
                        acc = acc + (rs_v[b, e, pl.ds(kk * L, L)]
                                     * rd_v[b, e, pl.ds(kk * L, L)])
                    plsc.store_scatter(tr_v, [lanes * 17 + t], acc)
                tot = plsc.load_gather(tr_v, [lanes])
                for l in range(1, L):
                    tot = tot + plsc.load_gather(tr_v, [lanes + l * 17])
                tot = jnp.where(tot > 0, tot, 0.2 * tot)
                lg_v[j, pl.ds(g * L, L)] = tot
                return carry

            lax.fori_loop(0, G, group, 0)

        for b in range(NB):
            start(b, b)

        def body(jo, carry):
            j0 = jo * NB
            for b in range(NB):
                j = j0 + b
                wait(b)
                compute(j, b)

                @pl.when(j + NB < NCH)
                def _():
                    start(j + NB, b)

            return carry

        lax.fori_loop(0, NCH // NB, body, 0)
        for b in range(NCH % NB):
            wait(b)
            compute(NCH - (NCH % NB) + b, b)
        pltpu.sync_copy(lg_v, out_hbm.at[wid])

    return k(h_flat, isrc3, idst3)


def _softmax_body(lg_ref, et_ref, o_ref):
    lg = lg_ref[...]
    et = et_ref[...]
    acc = jnp.zeros_like(lg)
    for r in range(R):
        m = et == r
        mr = jnp.max(jnp.where(m, lg, -jnp.inf))
        e = jnp.where(m, jnp.exp(lg - mr), 0.0)
        sr = jnp.sum(e)
        acc = acc + jnp.where(m, e / sr, 0.0)
    o_ref[...] = acc


def _softmax(logits2, et2):
    return pl.pallas_call(
        _softmax_body,
        out_shape=jax.ShapeDtypeStruct(logits2.shape, jnp.float32),
    )(logits2, et2)


def _pass2(h_flat, isrc3, dst3, attn3):
    @functools.partial(
        pl.kernel,
        out_type=jax.ShapeDtypeStruct((NC, N_NODES, D), jnp.float32),
        mesh=_sc_mesh(),
        scratch_types=[
            pltpu.VMEM((SUBCH, CH), jnp.int32),
            pltpu.VMEM((SUBCH, CH), jnp.int32),
            pltpu.VMEM((SUBCH, CH), jnp.float32),
            pltpu.VMEM((3, CH, D), jnp.float32),
            pltpu.VMEM_SHARED((N_NODES, D), jnp.float32),
            pltpu.SemaphoreType.DMA,
            pltpu.SemaphoreType.DMA,
            pltpu.SemaphoreType.DMA,
        ],
        compiler_params=pltpu.CompilerParams(needs_layout_passes=False),
    )
    def k(h_hbm, isrc_hbm, dst_hbm, attn_hbm, out_hbm,
          isrc_v, dst_v, attn_v, rows_v, acc_s, g0, g1, g2):
        cid = lax.axis_index("c")
        sid = lax.axis_index("s")
        wid = cid * NS + sid
        gsems = (g0, g1, g2)
        NB = 3

        zv = jnp.zeros((L,), jnp.float32)

        def zrow(i, carry):
            for kk in range(KV):
                rows_v[0, i, pl.ds(kk * L, L)] = zv
            return carry

        lax.fori_loop(0, CH, zrow, 0)
        for b in range(WB_ROWS // CH):
            pltpu.sync_copy(
                rows_v.at[0], acc_s.at[pl.ds(sid * WB_STRIDE + b * CH, CH)])
        plsc.subcore_barrier()

        def g_start(j, b):
            pltpu.async_copy(h_hbm.at[isrc_v.at[j]], rows_v.at[b], gsems[b])

        def g_wait(b):
            pltpu.make_async_copy(
                h_hbm.at[pl.ds(0, CH)], rows_v.at[b], gsems[b]).wait()

        def scale(j, b):
            def group(g, carry):
                a16 = attn_v[j, pl.ds(g * L, L)]
                for t in range(L):
                    e = g * L + t
                    ae = _lane_bcast(a16, t)
                    for kk in range(KV):
                        rows_v[b, e, pl.ds(kk * L, L)] = (
                            rows_v[b, e, pl.ds(kk * L, L)] * ae)
                return carry

            lax.fori_loop(0, G, group, 0)

        def supchunk(s, carry):
            pltpu.sync_copy(isrc_hbm.at[wid, s], isrc_v)
            pltpu.sync_copy(dst_hbm.at[wid, s], dst_v)
            pltpu.sync_copy(attn_hbm.at[wid, s], attn_v)
            for b in range(NB):
                g_start(b, b)

            def body(jo, carry2):
                j0 = jo * NB
                for b in range(NB):
                    j = j0 + b
                    g_wait(b)
                    scale(j, b)
                    pltpu.sync_copy(
                        rows_v.at[b], acc_s.at[dst_v.at[j]], add=True)

                    @pl.when(j + NB < SUBCH)
                    def _():
                        g_start(j + NB, b)

                return carry2

            lax.fori_loop(0, SUBCH // NB, body, 0)
            g_wait(0)
            scale(SUBCH - 1, 0)
            pltpu.sync_copy(
                rows_v.at[0], acc_s.at[dst_v.at[SUBCH - 1]], add=True)
            return carry

        lax.fori_loop(0, NSUP, supchunk, 0)
        plsc.subcore_barrier()
        pltpu.sync_copy(
            acc_s.at[pl.ds(sid * WB_STRIDE, WB_ROWS)],
            out_hbm.at[cid, pl.ds(sid * WB_STRIDE, WB_ROWS)])

    return k(h_flat, isrc3, dst3, attn3)


def _add_body(a_ref, b_ref, o_ref):
    o_ref[...] = a_ref[...] + b_ref[...]


def _final_add(a, b):
    BM = 1000
    return pl.pallas_call(
        _add_body,
        grid=(N_NODES // BM,),
        in_specs=[
            pl.BlockSpec((BM, D), lambda i: (i, 0)),
            pl.BlockSpec((BM, D), lambda i: (i, 0)),
        ],
        out_specs=pl.BlockSpec((BM, D), lambda i: (i, 0)),
        out_shape=jax.ShapeDtypeStruct((N_NODES, D), jnp.float32),
    )(a, b)


def kernel(x, edge_index, edge_type, W):
    src = edge_index[0]
    dst = edge_index[1]
    flat_src = (edge_type * N_NODES + src).reshape(NW, NCH, CH)
    flat_dst = (edge_type * N_NODES + dst).reshape(NW, NCH, CH)
    dst3 = dst.reshape(NW, NCH, CH)

    h = _compute_h(x, W).reshape(R * N_NODES, D)
    logits = _pass1(h, flat_src, flat_dst).reshape(N_EDGES)
    attn = _softmax(
        logits.reshape(N_EDGES // D, D),
        edge_type.reshape(N_EDGES // D, D),
    ).reshape(NW, NSUP, SUBCH, CH)
    parts = _pass2(h, flat_src.reshape(NW, NSUP, SUBCH, CH),
                   dst3.reshape(NW, NSUP, SUBCH, CH), attn)
    return _final_add(parts[0], parts[1])
